# Initial kernel scaffold; baseline (speedup 1.0000x reference)
import jax
import jax.numpy as jnp
from jax.experimental import pallas as pl


def kernel(values, remap_table, table):
    raise NotImplementedError("write your pallas kernel here")


# speedup vs baseline: 1.7729x; 1.7729x over previous
import functools

import jax
import jax.numpy as jnp
from jax import lax
from jax.experimental import pallas as pl
from jax.experimental.pallas import tpu as pltpu
from jax.experimental.pallas import tpu_sc as plsc

B = 16384
L = 20
DIM = 32
NC = 2
NS = 16
NW = NC * NS
IDS_W = B * L // NW
BAGS_W = B // NW
IW = 128
NJ = IDS_W // IW
CB = 64
CROWS = CB * L
CJ = CROWS // IW
NCHUNK = BAGS_W // CB


def kernel(values, remap_table, table):
    v = values.reshape(NW, NJ, IW)
    mesh = plsc.VectorSubcoreMesh(core_axis_name="c", subcore_axis_name="s")

    @functools.partial(
        pl.kernel,
        mesh=mesh,
        compiler_params=pltpu.CompilerParams(use_tc_tiling_on_sc=False),
        out_type=jax.ShapeDtypeStruct((B, DIM), jnp.float32),
        scratch_types=[
            pltpu.VMEM((NJ, IW), jnp.int32),
            pltpu.VMEM((NJ, IW), jnp.int32),
            pltpu.VMEM((CROWS, DIM), jnp.float32),
            pltpu.VMEM((CB, DIM), jnp.float32),
            pltpu.SemaphoreType.DMA,
            pltpu.SemaphoreType.DMA,
        ],
    )
    def k(v_hbm, remap_hbm, table_hbm, out_hbm, ids_v, rid_v, rows_v, out_v,
          sem_r, sem_t):
        wid = lax.axis_index("c") * NS + lax.axis_index("s")
        pltpu.sync_copy(v_hbm.at[wid], ids_v)
        rcps = [
            pltpu.async_copy(remap_hbm.at[ids_v.at[j]], rid_v.at[j], sem_r)
            for j in range(NJ)
        ]
        for cp in rcps:
            cp.wait()
        bag_base = wid * BAGS_W
        for c in range(NCHUNK):
            tcps = [
                pltpu.async_copy(
                    table_hbm.at[rid_v.at[c * CJ + j]],
                    rows_v.at[pl.ds(j * IW, IW)],
                    sem_t,
                )
                for j in range(CJ)
            ]
            for cp in tcps:
                cp.wait()

            def pool(b, carry):
                r0 = b * L
                a0 = rows_v[r0, pl.ds(0, 16)]
                a1 = rows_v[r0, pl.ds(16, 16)]
                for l in range(1, L):
                    a0 = a0 + rows_v[r0 + l, pl.ds(0, 16)]
                    a1 = a1 + rows_v[r0 + l, pl.ds(16, 16)]
                out_v[b, pl.ds(0, 16)] = a0
                out_v[b, pl.ds(16, 16)] = a1
                return carry

            lax.fori_loop(0, CB, pool, 0)
            pltpu.sync_copy(out_v, out_hbm.at[pl.ds(bag_base + c * CB, CB)])

    return k(v, remap_table, table)


# speedup vs baseline: 2.6160x; 1.4755x over previous
import functools

import jax
import jax.numpy as jnp
from jax import lax
from jax.experimental import pallas as pl
from jax.experimental.pallas import tpu as pltpu
from jax.experimental.pallas import tpu_sc as plsc

B = 16384
L = 20
DIM = 32
NEMB = 1000000
NC = 2
NS = 16
NW = NC * NS
IDS_W = B * L // NW
BAGS_W = B // NW
IW = 128
NJ = IDS_W // IW
CB = 64
CROWS = CB * L
CJ = CROWS // IW
NCHUNK = BAGS_W // CB

TCOL = 4096
QROWS = TCOL // 4
TGRID = -(-NEMB // TCOL)


def _transpose_block(in_ref, out_ref):
    y = in_ref[...].T
    out_ref[...] = jnp.concatenate(
        [y[q * QROWS:(q + 1) * QROWS] for q in range(4)], axis=1)


def _relayout_table(table):
    tt = table.T
    t128 = pl.pallas_call(
        _transpose_block,
        grid=(TGRID,),
        in_specs=[pl.BlockSpec((DIM, TCOL), lambda k: (0, k))],
        out_specs=pl.BlockSpec((QROWS, 128), lambda k: (k, 0)),
        out_shape=jax.ShapeDtypeStruct((TGRID * QROWS, 128), jnp.float32),
    )(tt)
    return t128.reshape(TGRID * TCOL, DIM)


def kernel(values, remap_table, table):
    v = values.reshape(NW, NJ, IW)
    t32 = _relayout_table(table)
    mesh = plsc.VectorSubcoreMesh(core_axis_name="c", subcore_axis_name="s")

    @functools.partial(
        pl.kernel,
        mesh=mesh,
        compiler_params=pltpu.CompilerParams(use_tc_tiling_on_sc=False),
        out_type=jax.ShapeDtypeStruct((B, DIM), jnp.float32),
        scratch_types=[
            pltpu.VMEM((NJ, IW), jnp.int32),
            pltpu.VMEM((NJ, IW), jnp.int32),
            pltpu.VMEM((CROWS, DIM), jnp.float32),
            pltpu.VMEM((CB, DIM), jnp.float32),
            pltpu.SemaphoreType.DMA,
            pltpu.SemaphoreType.DMA,
        ],
    )
    def k(v_hbm, remap_hbm, table_hbm, out_hbm, ids_v, rid_v, rows_v, out_v,
          sem_r, sem_t):
        wid = lax.axis_index("c") * NS + lax.axis_index("s")
        pltpu.sync_copy(v_hbm.at[wid], ids_v)
        rcps = [
            pltpu.async_copy(remap_hbm.at[ids_v.at[j]], rid_v.at[j], sem_r)
            for j in range(NJ)
        ]
        for cp in rcps:
            cp.wait()

        def xform(t, carry):
            j = t // (IW // 16)
            s = (t % (IW // 16)) * 16
            r = rid_v[j, pl.ds(s, 16)]
            vv = ((r & (-TCOL)) + ((r & (QROWS - 1)) << 2)
                  + ((r >> 10) & 3))
            ids_v[j, pl.ds(s, 16)] = vv
            return carry

        lax.fori_loop(0, NJ * (IW // 16), xform, 0)

        bag_base = wid * BAGS_W
        for c in range(NCHUNK):
            tcps = [
                pltpu.async_copy(
                    table_hbm.at[ids_v.at[c * CJ + j]],
                    rows_v.at[pl.ds(j * IW, IW)],
                    sem_t,
                )
                for j in range(CJ)
            ]
            for cp in tcps:
                cp.wait()

            def pool(b, carry):
                r0 = b * L
                a0 = rows_v[r0, pl.ds(0, 16)]
                a1 = rows_v[r0, pl.ds(16, 16)]
                for l in range(1, L):
                    a0 = a0 + rows_v[r0 + l, pl.ds(0, 16)]
                    a1 = a1 + rows_v[r0 + l, pl.ds(16, 16)]
                out_v[b, pl.ds(0, 16)] = a0
                out_v[b, pl.ds(16, 16)] = a1
                return carry

            lax.fori_loop(0, CB, pool, 0)
            pltpu.sync_copy(out_v, out_hbm.at[pl.ds(bag_base + c * CB, CB)])

    return k(v, remap_table, t32)


# speedup vs baseline: 3.3367x; 1.2755x over previous
import functools

import jax
import jax.numpy as jnp
from jax import lax
from jax.experimental import pallas as pl
from jax.experimental.pallas import tpu as pltpu
from jax.experimental.pallas import tpu_sc as plsc

B = 16384
L = 20
DIM = 32
NEMB = 1000000
NC = 2
NS = 16
NW = NC * NS
IDS_W = B * L // NW
BAGS_W = B // NW
IW = 128
NJ = IDS_W // IW
CB = 64
CROWS = CB * L
CJ = CROWS // IW
NCHUNK = BAGS_W // CB

TCOL = 4096
QROWS = TCOL // 4
TGRID = -(-NEMB // TCOL)


def _transpose_block(in_ref, out_ref):
    x = in_ref[...]
    tail = NEMB - (TGRID - 1) * TCOL
    x = lax.cond(
        pl.program_id(0) == TGRID - 1,
        lambda x: x * (lax.broadcasted_iota(jnp.int32, (DIM, TCOL), 1)
                       < tail).astype(jnp.float32),
        lambda x: x,
        x,
    )
    xx = jnp.concatenate(
        [x[:, q * QROWS:(q + 1) * QROWS] for q in range(4)], axis=0)
    out_ref[...] = xx.T


def _relayout_table(table):
    tt = table.T
    t128 = pl.pallas_call(
        _transpose_block,
        grid=(TGRID,),
        in_specs=[pl.BlockSpec((DIM, TCOL), lambda k: (0, k))],
        out_specs=pl.BlockSpec((QROWS, 128), lambda k: (k, 0)),
        out_shape=jax.ShapeDtypeStruct((TGRID * QROWS, 128), jnp.float32),
        compiler_params=pltpu.CompilerParams(fuse_transposed_lhs_in_matmul=True),
    )(tt)
    return t128.reshape(TGRID * TCOL, DIM)


def kernel(values, remap_table, table):
    v = values.reshape(NW, NJ, IW)
    t32 = _relayout_table(table)
    mesh = plsc.VectorSubcoreMesh(core_axis_name="c", subcore_axis_name="s")

    @functools.partial(
        pl.kernel,
        mesh=mesh,
        compiler_params=pltpu.CompilerParams(use_tc_tiling_on_sc=False),
        out_type=jax.ShapeDtypeStruct((B, DIM), jnp.float32),
        scratch_types=[
            pltpu.VMEM((NJ, IW), jnp.int32),
            pltpu.VMEM((NJ, IW), jnp.int32),
            pltpu.VMEM((CROWS, DIM), jnp.float32),
            pltpu.VMEM((CB, DIM), jnp.float32),
            pltpu.SemaphoreType.DMA,
            pltpu.SemaphoreType.DMA,
        ],
    )
    def k(v_hbm, remap_hbm, table_hbm, out_hbm, ids_v, rid_v, rows_v, out_v,
          sem_r, sem_t):
        wid = lax.axis_index("c") * NS + lax.axis_index("s")
        pltpu.sync_copy(v_hbm.at[wid], ids_v)
        rcps = [
            pltpu.async_copy(remap_hbm.at[ids_v.at[j]], rid_v.at[j], sem_r)
            for j in range(NJ)
        ]
        for cp in rcps:
            cp.wait()

        def xform(t, carry):
            j = t // (IW // 16)
            s = (t % (IW // 16)) * 16
            r = rid_v[j, pl.ds(s, 16)]
            vv = ((r & (-TCOL)) + ((r & (QROWS - 1)) << 2)
                  + ((r >> 10) & 3))
            ids_v[j, pl.ds(s, 16)] = vv
            return carry

        lax.fori_loop(0, NJ * (IW // 16), xform, 0)

        bag_base = wid * BAGS_W
        for c in range(NCHUNK):
            tcps = [
                pltpu.async_copy(
                    table_hbm.at[ids_v.at[c * CJ + j]],
                    rows_v.at[pl.ds(j * IW, IW)],
                    sem_t,
                )
                for j in range(CJ)
            ]
            for cp in tcps:
                cp.wait()

            def pool(b, carry):
                r0 = b * L
                a0 = rows_v[r0, pl.ds(0, 16)]
                a1 = rows_v[r0, pl.ds(16, 16)]
                for l in range(1, L):
                    a0 = a0 + rows_v[r0 + l, pl.ds(0, 16)]
                    a1 = a1 + rows_v[r0 + l, pl.ds(16, 16)]
                out_v[b, pl.ds(0, 16)] = a0
                out_v[b, pl.ds(16, 16)] = a1
                return carry

            lax.fori_loop(0, CB, pool, 0)
            pltpu.sync_copy(out_v, out_hbm.at[pl.ds(bag_base + c * CB, CB)])

    return k(v, remap_table, t32)


# speedup vs baseline: 4.6634x; 1.3976x over previous
import functools

import jax
import jax.numpy as jnp
from jax import lax
from jax.experimental import pallas as pl
from jax.experimental.pallas import tpu as pltpu
from jax.experimental.pallas import tpu_sc as plsc

B = 16384
L = 20
DIM = 32
NEMB = 1000000
NC = 2
NS = 16
NW = NC * NS
IDS_W = B * L // NW
BAGS_W = B // NW
IW = 128
NJ = IDS_W // IW
CB = 64
CROWS = CB * L
CJ = CROWS // IW
NCHUNK = BAGS_W // CB

TCOL = 16384
QROWS = TCOL // 4
QSHIFT = QROWS.bit_length() - 1
TGRID = -(-NEMB // TCOL)


def _transpose_block(in_ref, out_ref):
    x = in_ref[...]
    tail = NEMB - (TGRID - 1) * TCOL
    x = lax.cond(
        pl.program_id(0) == TGRID - 1,
        lambda x: x * (lax.broadcasted_iota(jnp.int32, (DIM, TCOL), 1)
                       < tail).astype(jnp.float32),
        lambda x: x,
        x,
    )
    xx = jnp.concatenate(
        [x[:, q * QROWS:(q + 1) * QROWS] for q in range(4)], axis=0)
    out_ref[...] = xx.T


def _relayout_table(table):
    tt = table.T
    t128 = pl.pallas_call(
        _transpose_block,
        grid=(TGRID,),
        in_specs=[pl.BlockSpec((DIM, TCOL), lambda k: (0, k))],
        out_specs=pl.BlockSpec((QROWS, 128), lambda k: (k, 0)),
        out_shape=jax.ShapeDtypeStruct((TGRID * QROWS, 128), jnp.float32),
        compiler_params=pltpu.CompilerParams(fuse_transposed_lhs_in_matmul=True),
    )(tt)
    return t128.reshape(TGRID * TCOL, DIM)


def kernel(values, remap_table, table):
    v = values.reshape(NW, NJ, IW)
    t32 = _relayout_table(table)
    mesh = plsc.VectorSubcoreMesh(core_axis_name="c", subcore_axis_name="s")

    @functools.partial(
        pl.kernel,
        mesh=mesh,
        compiler_params=pltpu.CompilerParams(use_tc_tiling_on_sc=False),
        out_type=jax.ShapeDtypeStruct((B, DIM), jnp.float32),
        scratch_types=[
            pltpu.VMEM((NJ, IW), jnp.int32),
            pltpu.VMEM((NJ, IW), jnp.int32),
            pltpu.VMEM((CROWS, DIM), jnp.float32),
            pltpu.VMEM((CB, DIM), jnp.float32),
            pltpu.SemaphoreType.DMA,
            pltpu.SemaphoreType.DMA,
        ],
    )
    def k(v_hbm, remap_hbm, table_hbm, out_hbm, ids_v, rid_v, rows_v, out_v,
          sem_r, sem_t):
        wid = lax.axis_index("c") * NS + lax.axis_index("s")
        pltpu.sync_copy(v_hbm.at[wid], ids_v)
        rcps = [
            pltpu.async_copy(remap_hbm.at[ids_v.at[j]], rid_v.at[j], sem_r)
            for j in range(NJ)
        ]
        for cp in rcps:
            cp.wait()

        def xform(t, carry):
            j = t // (IW // 16)
            s = (t % (IW // 16)) * 16
            r = rid_v[j, pl.ds(s, 16)]
            vv = ((r & (-TCOL)) + ((r & (QROWS - 1)) << 2)
                  + ((r >> QSHIFT) & 3))
            ids_v[j, pl.ds(s, 16)] = vv
            return carry

        lax.fori_loop(0, NJ * (IW // 16), xform, 0)

        bag_base = wid * BAGS_W
        for c in range(NCHUNK):
            tcps = [
                pltpu.async_copy(
                    table_hbm.at[ids_v.at[c * CJ + j]],
                    rows_v.at[pl.ds(j * IW, IW)],
                    sem_t,
                )
                for j in range(CJ)
            ]
            for cp in tcps:
                cp.wait()

            def pool(b, carry):
                r0 = b * L
                a0 = rows_v[r0, pl.ds(0, 16)]
                a1 = rows_v[r0, pl.ds(16, 16)]
                for l in range(1, L):
                    a0 = a0 + rows_v[r0 + l, pl.ds(0, 16)]
                    a1 = a1 + rows_v[r0 + l, pl.ds(16, 16)]
                out_v[b, pl.ds(0, 16)] = a0
                out_v[b, pl.ds(16, 16)] = a1
                return carry

            lax.fori_loop(0, CB, pool, 0)
            pltpu.sync_copy(out_v, out_hbm.at[pl.ds(bag_base + c * CB, CB)])

    return k(v, remap_table, t32)
